# baseline (speedup 1.0000x reference)
import jax
import jax.numpy as jnp
from jax import lax
from jax.experimental import pallas as pl
from jax.experimental.pallas import tpu as pltpu
from jax.experimental.pallas import tpu_sc as plsc

ROWS = 16384
COLS = 2048
LANES_SC = 16
SUB = COLS // LANES_SC
NUM_WORKERS = 32
ROWS_PER_W = ROWS // NUM_WORKERS
CHUNK_ROWS = 16
CHUNK_SUBS = CHUNK_ROWS * SUB
PAIRS = CHUNK_ROWS * (SUB // 2)
N_CHUNKS = ROWS_PER_W // CHUNK_ROWS


def _sc_flip(in_hbm, out_hbm, v0, v1, sin0, sin1, sout0, sout1):
    c = lax.axis_index("c")
    s = lax.axis_index("s")
    wid = s * 2 + c
    base = wid * (ROWS_PER_W * SUB)
    bufs = (v0, v1)
    sins = (sin0, sin1)
    souts = (sout0, sout1)

    def off(ci):
        return base + ci * CHUNK_SUBS

    pltpu.async_copy(in_hbm.at[pl.ds(off(0), CHUNK_SUBS)], v0, sin0)

    def outer(g, carry):
        for b in range(2):
            ci = 2 * g + b
            nb = 1 - b

            @pl.when(ci >= 1)
            def _():
                pltpu.make_async_copy(
                    bufs[nb],
                    out_hbm.at[pl.ds(off(ci - 1), CHUNK_SUBS)],
                    souts[nb],
                ).wait()

            @pl.when(ci + 1 < N_CHUNKS)
            def _():
                pltpu.async_copy(
                    in_hbm.at[pl.ds(off(ci + 1), CHUNK_SUBS)],
                    bufs[nb], sins[nb],
                )

            pltpu.make_async_copy(
                in_hbm.at[pl.ds(off(ci), CHUNK_SUBS)], bufs[b], sins[b]
            ).wait()

            @plsc.parallel_loop(0, PAIRS, unroll=8)
            def _(j):
                r = j >> 6
                k = j & (SUB // 2 - 1)
                a = (r << 7) + k
                bo = (r << 7) + (SUB - 1 - k)
                x = jnp.flip(bufs[b][bo], axis=0)
                y = jnp.flip(bufs[b][a], axis=0)
                bufs[b][a] = x
                bufs[b][bo] = y

            pltpu.async_copy(
                bufs[b], out_hbm.at[pl.ds(off(ci), CHUNK_SUBS)], souts[b]
            )
        return carry

    lax.fori_loop(0, N_CHUNKS // 2, outer, 0)

    pltpu.make_async_copy(
        bufs[1], out_hbm.at[pl.ds(off(N_CHUNKS - 1), CHUNK_SUBS)], souts[1]
    ).wait()


def kernel(inputs, cond_inputs):
    flat_in = inputs.reshape(ROWS * SUB, LANES_SC)
    mesh = plsc.VectorSubcoreMesh(core_axis_name="c", subcore_axis_name="s")
    f = pl.kernel(
        _sc_flip,
        mesh=mesh,
        out_type=jax.ShapeDtypeStruct((ROWS * SUB, LANES_SC), jnp.float32),
        compiler_params=pltpu.CompilerParams(
            needs_layout_passes=False, use_tc_tiling_on_sc=False
        ),
        scratch_types=[
            pltpu.VMEM((CHUNK_SUBS, LANES_SC), jnp.float32),
            pltpu.VMEM((CHUNK_SUBS, LANES_SC), jnp.float32),
            pltpu.SemaphoreType.DMA,
            pltpu.SemaphoreType.DMA,
            pltpu.SemaphoreType.DMA,
            pltpu.SemaphoreType.DMA,
        ],
    )
    out = f(flat_in)
    return (out.reshape(ROWS, COLS), 0.0)


# speedup vs baseline: 3.9642x; 3.9642x over previous
import jax
import jax.numpy as jnp
from jax.experimental import pallas as pl

ROWS = 16384
COLS = 2048
BLOCK_ROWS = 2048
BLOCK_COLS = 1024
LANES = 128
NUM_SUB = BLOCK_COLS // LANES
NUM_CB = COLS // BLOCK_COLS


def _flip_block(in_ref, out_ref):
    rev = (LANES - 1) - jax.lax.broadcasted_iota(
        jnp.int32, (BLOCK_ROWS, LANES), 1
    )
    for j in range(NUM_SUB):
        src = NUM_SUB - 1 - j
        x = in_ref[:, src * LANES:(src + 1) * LANES]
        out_ref[:, j * LANES:(j + 1) * LANES] = jnp.take_along_axis(
            x, rev, axis=1
        )


def kernel(inputs, cond_inputs):
    out = pl.pallas_call(
        _flip_block,
        grid=(ROWS // BLOCK_ROWS, NUM_CB),
        in_specs=[
            pl.BlockSpec(
                (BLOCK_ROWS, BLOCK_COLS), lambda i, j: (i, NUM_CB - 1 - j)
            )
        ],
        out_specs=pl.BlockSpec((BLOCK_ROWS, BLOCK_COLS), lambda i, j: (i, j)),
        out_shape=jax.ShapeDtypeStruct((ROWS, COLS), inputs.dtype),
    )(inputs)
    return (out, 0.0)
